# baseline (speedup 1.0000x reference)
import functools

import jax
import jax.numpy as jnp
from jax import lax
from jax.experimental import pallas as pl
from jax.experimental.pallas import tpu as pltpu
from jax.experimental.pallas import tpu_sc as plsc

B, S, D, DFF, E, TOPK = 2, 2048, 1024, 4096, 16, 2
T = B * S
NSLOT = T * TOPK
TILE_M = 256
NT = (NSLOT + E * (TILE_M - 1) + TILE_M - 1) // TILE_M
P = NT * TILE_M
F_TILE = 512
NF = DFF // F_TILE
RT = 1024
LANES = 128

NW = 32
RW = P // NW
GCH = 48
TW = T // NW
CCH = 32


def _router_body(x_ref, wr_ref, br_ref, g0_ref, g1_ref, i0_ref, i1_ref):
    logits = jnp.dot(x_ref[...], wr_ref[...], preferred_element_type=jnp.float32)
    logits = logits + br_ref[...]
    lanes = lax.broadcasted_iota(jnp.int32, logits.shape, 1)
    neg = jnp.float32(float("-inf"))
    logits = jnp.where(lanes < E, logits, neg)
    m0 = jnp.max(logits, axis=1, keepdims=True)
    i0 = jnp.min(jnp.where(logits == m0, lanes, E), axis=1, keepdims=True)
    masked = jnp.where(lanes == i0, neg, logits)
    m1 = jnp.max(masked, axis=1, keepdims=True)
    i1 = jnp.min(jnp.where(masked == m1, lanes, E), axis=1, keepdims=True)
    eexp = jnp.exp(m1 - m0)
    denom = 1.0 + eexp
    g0_ref[...] = 1.0 / denom
    g1_ref[...] = eexp / denom
    i0_ref[...] = i0
    i1_ref[...] = i1


def _router(x_flat, wr_pad, br_pad):
    return pl.pallas_call(
        _router_body,
        grid=(T // RT,),
        in_specs=[
            pl.BlockSpec((RT, D), lambda m: (m, 0)),
            pl.BlockSpec((D, LANES), lambda m: (0, 0)),
            pl.BlockSpec((1, LANES), lambda m: (0, 0)),
        ],
        out_specs=[
            pl.BlockSpec((RT, 1), lambda m: (m, 0)),
            pl.BlockSpec((RT, 1), lambda m: (m, 0)),
            pl.BlockSpec((RT, 1), lambda m: (m, 0)),
            pl.BlockSpec((RT, 1), lambda m: (m, 0)),
        ],
        out_shape=[
            jax.ShapeDtypeStruct((T, 1), jnp.float32),
            jax.ShapeDtypeStruct((T, 1), jnp.float32),
            jax.ShapeDtypeStruct((T, 1), jnp.int32),
            jax.ShapeDtypeStruct((T, 1), jnp.int32),
        ],
    )(x_flat, wr_pad, br_pad)


@functools.cache
def _sc_kernels():
    mesh = plsc.VectorSubcoreMesh(core_axis_name="c", subcore_axis_name="s")

    @functools.partial(
        pl.kernel,
        out_type=jax.ShapeDtypeStruct((P, D), jnp.float32),
        mesh=mesh,
        scratch_types=[
            pltpu.VMEM((RW,), jnp.int32),
            pltpu.VMEM((GCH, D), jnp.float32),
            pltpu.VMEM((GCH, D), jnp.float32),
            pltpu.SemaphoreType.DMA,
            pltpu.SemaphoreType.DMA,
        ],
    )
    def sc_gather(x_hbm, st_hbm, out_hbm, idx_v, rows0_v, rows1_v, sem0, sem1):
        wid = lax.axis_index("s") * 2 + lax.axis_index("c")
        base = wid * RW
        pltpu.sync_copy(st_hbm.at[pl.ds(base, RW)], idx_v)
        nch = RW // GCH
        bufs = (rows0_v, rows1_v)
        sems = (sem0, sem1)
        cps = [None, None]
        cps[0] = pltpu.async_copy(x_hbm.at[idx_v.at[pl.ds(0, GCH)]], rows0_v, sem0)
        for c in range(1, nch):
            cps[c % 2] = pltpu.async_copy(
                x_hbm.at[idx_v.at[pl.ds(c * GCH, GCH)]], bufs[c % 2], sems[c % 2]
            )
            cps[(c - 1) % 2].wait()
            pltpu.sync_copy(
                bufs[(c - 1) % 2], out_hbm.at[pl.ds(base + (c - 1) * GCH, GCH)]
            )
        cps[(nch - 1) % 2].wait()
        pltpu.sync_copy(
            bufs[(nch - 1) % 2], out_hbm.at[pl.ds(base + (nch - 1) * GCH, GCH)]
        )

    @functools.partial(
        pl.kernel,
        out_type=jax.ShapeDtypeStruct((T, D), jnp.float32),
        mesh=mesh,
        scratch_types=[
            pltpu.VMEM((TW,), jnp.int32),
            pltpu.VMEM((TW,), jnp.int32),
            pltpu.VMEM((CCH, D), jnp.float32),
            pltpu.VMEM((CCH, D), jnp.float32),
            pltpu.SemaphoreType.DMA,
            pltpu.SemaphoreType.DMA,
        ],
    )
    def sc_combine(yg_hbm, pp0_hbm, pp1_hbm, out_hbm, i0_v, i1_v, r0_v, r1_v,
                   s0, s1):
        wid = lax.axis_index("s") * 2 + lax.axis_index("c")
        base = wid * TW
        pltpu.sync_copy(pp0_hbm.at[pl.ds(base, TW)], i0_v)
        pltpu.sync_copy(pp1_hbm.at[pl.ds(base, TW)], i1_v)
        for c in range(TW // CCH):
            cp0 = pltpu.async_copy(
                yg_hbm.at[i0_v.at[pl.ds(c * CCH, CCH)]], r0_v, s0
            )
            cp1 = pltpu.async_copy(
                yg_hbm.at[i1_v.at[pl.ds(c * CCH, CCH)]], r1_v, s1
            )
            cp0.wait()
            cp1.wait()

            def _add_row(i, carry):
                for j in range(D // 16):
                    sl = pl.ds(j * 16, 16)
                    r0_v[i, sl] = r0_v[i, sl] + r1_v[i, sl]
                return carry

            lax.fori_loop(0, CCH, _add_row, 0)
            pltpu.sync_copy(r0_v, out_hbm.at[pl.ds(base + c * CCH, CCH)])

    return sc_gather, sc_combine


def _ffn_body(te_ref, xg_ref, w1_ref, b1_ref, w2_ref, b2_ref, gs_ref, out_ref):
    f = pl.program_id(1)
    xb = xg_ref[...].astype(jnp.bfloat16)
    h = jnp.dot(xb, w1_ref[0], preferred_element_type=jnp.float32)
    h = jnp.maximum(h + b1_ref[0], 0.0).astype(jnp.bfloat16)
    y = jnp.dot(h, w2_ref[0], preferred_element_type=jnp.float32)

    @pl.when(f == 0)
    def _():
        out_ref[...] = y + b2_ref[0]

    @pl.when(f > 0)
    def _():
        out_ref[...] += y

    @pl.when(f == NF - 1)
    def _():
        out_ref[...] *= gs_ref[...]


def _ffn(te, xg, W1, b1r, W2, b2r, gs):
    grid_spec = pltpu.PrefetchScalarGridSpec(
        num_scalar_prefetch=1,
        grid=(NT, NF),
        in_specs=[
            pl.BlockSpec((TILE_M, D), lambda m, f, te_ref: (m, 0)),
            pl.BlockSpec((1, D, F_TILE), lambda m, f, te_ref: (te_ref[m], 0, f)),
            pl.BlockSpec((1, 1, F_TILE), lambda m, f, te_ref: (te_ref[m], 0, f)),
            pl.BlockSpec((1, F_TILE, D), lambda m, f, te_ref: (te_ref[m], f, 0)),
            pl.BlockSpec((1, 1, D), lambda m, f, te_ref: (te_ref[m], 0, 0)),
            pl.BlockSpec((TILE_M, 1), lambda m, f, te_ref: (m, 0)),
        ],
        out_specs=pl.BlockSpec((TILE_M, D), lambda m, f, te_ref: (m, 0)),
    )
    return pl.pallas_call(
        _ffn_body,
        grid_spec=grid_spec,
        out_shape=jax.ShapeDtypeStruct((P, D), jnp.float32),
        compiler_params=pltpu.CompilerParams(
            dimension_semantics=("parallel", "arbitrary")
        ),
    )(te, xg, W1, b1r, W2, b2r, gs)


def _dispatch_indices(i0, i1, g0, g1):
    ids = jnp.concatenate([i0, i1], axis=1).reshape(-1)
    gsl = jnp.concatenate([g0, g1], axis=1).reshape(-1)
    onehot = (ids[:, None] == jnp.arange(E, dtype=jnp.int32)[None, :]).astype(
        jnp.int32
    )
    csum = jnp.cumsum(onehot, axis=0)
    rank = jnp.take_along_axis(csum, ids[:, None], axis=1)[:, 0] - 1
    counts = csum[-1]
    tiles_pe = (counts + TILE_M - 1) // TILE_M
    bounds = jnp.cumsum(tiles_pe)
    tile_start = jnp.concatenate([jnp.zeros((1,), bounds.dtype), bounds[:-1]])
    pp = (tile_start[ids] * TILE_M + rank).astype(jnp.int32)
    st = (
        jnp.zeros((P,), jnp.int32)
        .at[pp]
        .set(jnp.arange(NSLOT, dtype=jnp.int32) // TOPK)
    )
    gs = jnp.zeros((P, 1), jnp.float32).at[pp, 0].set(gsl)
    m_ids = jnp.arange(NT, dtype=jnp.int32)
    te = jnp.minimum(
        jnp.sum((m_ids[:, None] >= bounds[None, :]).astype(jnp.int32), axis=1),
        E - 1,
    ).astype(jnp.int32)
    return pp, st, gs, te


@jax.jit
def kernel(x, Wr, br, W1, b1, W2, b2):
    x_flat = x.reshape(T, D)
    wr_pad = jnp.zeros((D, LANES), jnp.float32).at[:, :E].set(Wr)
    br_pad = jnp.zeros((1, LANES), jnp.float32).at[0, :E].set(br)
    g0, g1, i0, i1 = _router(x_flat, wr_pad, br_pad)
    pp, st, gs, te = _dispatch_indices(i0, i1, g0, g1)
    sc_gather, sc_combine = _sc_kernels()
    xg = sc_gather(x_flat, st)
    yg = _ffn(
        te,
        xg,
        W1.astype(jnp.bfloat16),
        b1.reshape(E, 1, DFF),
        W2.astype(jnp.bfloat16),
        b2.reshape(E, 1, D),
        gs,
    )
    pp2 = pp.reshape(T, TOPK)
    out_flat = sc_combine(yg, pp2[:, 0], pp2[:, 1])
    return out_flat.reshape(B, S, D)


# speedup vs baseline: 2.3330x; 2.3330x over previous
import functools

import jax
import jax.numpy as jnp
from jax import lax
from jax.experimental import pallas as pl
from jax.experimental.pallas import tpu as pltpu
from jax.experimental.pallas import tpu_sc as plsc

B, S, D, DFF, E, TOPK = 2, 2048, 1024, 4096, 16, 2
T = B * S
NSLOT = T * TOPK
TILE_M = 512
NT = (NSLOT + E * (TILE_M - 1) + TILE_M - 1) // TILE_M
P = NT * TILE_M
DHALF = DFF // 2
F_TILE = 512
NFH = DHALF // F_TILE
RT = 1024
LANES = 128

NW = 32
RW = P // NW
GCH = 64
TW = T // NW
CCH = 32


def _router_body(x_ref, wr_ref, br_ref, g0_ref, g1_ref, i0_ref, i1_ref):
    logits = jnp.dot(x_ref[...], wr_ref[...], preferred_element_type=jnp.float32)
    logits = logits + br_ref[...]
    lanes = lax.broadcasted_iota(jnp.int32, logits.shape, 1)
    neg = jnp.float32(float("-inf"))
    logits = jnp.where(lanes < E, logits, neg)
    m0 = jnp.max(logits, axis=1, keepdims=True)
    i0 = jnp.min(jnp.where(logits == m0, lanes, E), axis=1, keepdims=True)
    masked = jnp.where(lanes == i0, neg, logits)
    m1 = jnp.max(masked, axis=1, keepdims=True)
    i1 = jnp.min(jnp.where(masked == m1, lanes, E), axis=1, keepdims=True)
    eexp = jnp.exp(m1 - m0)
    denom = 1.0 + eexp
    g0_ref[...] = 1.0 / denom
    g1_ref[...] = eexp / denom
    i0_ref[...] = i0
    i1_ref[...] = i1


def _router(x_flat, wr_pad, br_pad):
    return pl.pallas_call(
        _router_body,
        grid=(T // RT,),
        in_specs=[
            pl.BlockSpec((RT, D), lambda m: (m, 0)),
            pl.BlockSpec((D, LANES), lambda m: (0, 0)),
            pl.BlockSpec((1, LANES), lambda m: (0, 0)),
        ],
        out_specs=[
            pl.BlockSpec((RT, 1), lambda m: (m, 0)),
            pl.BlockSpec((RT, 1), lambda m: (m, 0)),
            pl.BlockSpec((RT, 1), lambda m: (m, 0)),
            pl.BlockSpec((RT, 1), lambda m: (m, 0)),
        ],
        out_shape=[
            jax.ShapeDtypeStruct((T, 1), jnp.float32),
            jax.ShapeDtypeStruct((T, 1), jnp.float32),
            jax.ShapeDtypeStruct((T, 1), jnp.int32),
            jax.ShapeDtypeStruct((T, 1), jnp.int32),
        ],
    )(x_flat, wr_pad, br_pad)


@functools.cache
def _sc_kernels():
    mesh = plsc.VectorSubcoreMesh(core_axis_name="c", subcore_axis_name="s")

    @functools.partial(
        pl.kernel,
        out_type=jax.ShapeDtypeStruct((P, D), jnp.float32),
        mesh=mesh,
        scratch_types=[
            pltpu.VMEM((RW,), jnp.int32),
            pltpu.VMEM((GCH, D), jnp.float32),
            pltpu.SemaphoreType.DMA,
        ],
    )
    def sc_gather(x_hbm, st_hbm, out_hbm, idx_v, rows_v, sem):
        wid = lax.axis_index("s") * 2 + lax.axis_index("c")
        base = wid * RW
        pltpu.sync_copy(st_hbm.at[pl.ds(base, RW)], idx_v)
        for c in range(RW // GCH):
            pltpu.async_copy(
                x_hbm.at[idx_v.at[pl.ds(c * GCH, GCH)]], rows_v, sem
            ).wait()
            pltpu.sync_copy(rows_v, out_hbm.at[pl.ds(base + c * GCH, GCH)])

    @functools.partial(
        pl.kernel,
        out_type=jax.ShapeDtypeStruct((T, D), jnp.float32),
        mesh=mesh,
        scratch_types=[
            pltpu.VMEM((TW,), jnp.int32),
            pltpu.VMEM((TW,), jnp.int32),
            pltpu.VMEM((CCH, D), jnp.float32),
            pltpu.VMEM((CCH, D), jnp.float32),
            pltpu.SemaphoreType.DMA,
            pltpu.SemaphoreType.DMA,
        ],
    )
    def sc_combine(yg_hbm, pp0_hbm, pp1_hbm, out_hbm, i0_v, i1_v, r0_v, r1_v,
                   s0, s1):
        wid = lax.axis_index("s") * 2 + lax.axis_index("c")
        base = wid * TW
        pltpu.sync_copy(pp0_hbm.at[pl.ds(base, TW)], i0_v)
        pltpu.sync_copy(pp1_hbm.at[pl.ds(base, TW)], i1_v)
        for c in range(TW // CCH):
            cp0 = pltpu.async_copy(
                yg_hbm.at[i0_v.at[pl.ds(c * CCH, CCH)]], r0_v, s0
            )
            cp1 = pltpu.async_copy(
                yg_hbm.at[i1_v.at[pl.ds(c * CCH, CCH)]], r1_v, s1
            )
            cp0.wait()
            cp1.wait()

            def _add_row(i, carry):
                for j in range(D // 16):
                    sl = pl.ds(j * 16, 16)
                    r0_v[i, sl] = r0_v[i, sl] + r1_v[i, sl]
                return carry

            lax.fori_loop(0, CCH, _add_row, 0)
            pltpu.sync_copy(r0_v, out_hbm.at[pl.ds(base + c * CCH, CCH)])

    return sc_gather, sc_combine


def _ffn_body(sp_ref, xg_ref, w1_ref, b1_ref, w2_ref, b2_ref, gs_ref, out_ref):
    m = pl.program_id(0)
    f = pl.program_id(1)
    tu = sp_ref[NT]

    @pl.when(m < tu)
    def _():
        x = xg_ref[...]
        acc = None
        for j in range(NFH):
            fs = pl.ds(j * F_TILE, F_TILE)
            h = jnp.dot(
                x, w1_ref[0, :, fs], preferred_element_type=jnp.float32
            )
            h = jnp.maximum(h + b1_ref[0, :, fs], 0.0)
            y = jnp.dot(
                h, w2_ref[0, fs, :], preferred_element_type=jnp.float32
            )
            acc = y if acc is None else acc + y

        @pl.when(f == 0)
        def _():
            out_ref[...] = acc + b2_ref[0]

        @pl.when(f == 1)
        def _():
            out_ref[...] = (out_ref[...] + acc) * gs_ref[...]


def _ffn(sp, xg, W1, b1r, W2, b2r, gs):
    def _mclamp(m, sp_ref):
        return jnp.minimum(m, sp_ref[NT] - 1)

    def _snake(m, f):
        return jnp.where(m % 2 == 0, f, 1 - f)

    grid_spec = pltpu.PrefetchScalarGridSpec(
        num_scalar_prefetch=1,
        grid=(NT, 2),
        in_specs=[
            pl.BlockSpec(
                (TILE_M, D), lambda m, f, sp_ref: (_mclamp(m, sp_ref), 0)
            ),
            pl.BlockSpec(
                (1, D, DHALF), lambda m, f, sp_ref: (sp_ref[m], 0, _snake(m, f))
            ),
            pl.BlockSpec(
                (1, 1, DHALF), lambda m, f, sp_ref: (sp_ref[m], 0, _snake(m, f))
            ),
            pl.BlockSpec(
                (1, DHALF, D), lambda m, f, sp_ref: (sp_ref[m], _snake(m, f), 0)
            ),
            pl.BlockSpec((1, 1, D), lambda m, f, sp_ref: (sp_ref[m], 0, 0)),
            pl.BlockSpec(
                (TILE_M, 1), lambda m, f, sp_ref: (_mclamp(m, sp_ref), 0)
            ),
        ],
        out_specs=pl.BlockSpec(
            (TILE_M, D), lambda m, f, sp_ref: (_mclamp(m, sp_ref), 0)
        ),
    )
    return pl.pallas_call(
        _ffn_body,
        grid_spec=grid_spec,
        out_shape=jax.ShapeDtypeStruct((P, D), jnp.float32),
        compiler_params=pltpu.CompilerParams(
            dimension_semantics=("arbitrary", "arbitrary"),
        ),
    )(sp, xg, W1, b1r, W2, b2r, gs)


def _dispatch_indices(i0, i1, g0, g1):
    ids = jnp.concatenate([i0, i1], axis=1).reshape(-1)
    gsl = jnp.concatenate([g0, g1], axis=1).reshape(-1)
    onehot = (ids[:, None] == jnp.arange(E, dtype=jnp.int32)[None, :]).astype(
        jnp.int32
    )
    csum = jnp.cumsum(onehot, axis=0)
    rank = jnp.take_along_axis(csum, ids[:, None], axis=1)[:, 0] - 1
    counts = csum[-1]
    tiles_pe = (counts + TILE_M - 1) // TILE_M
    bounds = jnp.cumsum(tiles_pe)
    tile_start = jnp.concatenate([jnp.zeros((1,), bounds.dtype), bounds[:-1]])
    pp = (tile_start[ids] * TILE_M + rank).astype(jnp.int32)
    st = (
        (jnp.arange(P, dtype=jnp.int32) % T)
        .at[pp]
        .set(jnp.arange(NSLOT, dtype=jnp.int32) // TOPK)
    )
    gs = jnp.zeros((P, 1), jnp.float32).at[pp, 0].set(gsl)
    m_ids = jnp.arange(NT, dtype=jnp.int32)
    te = jnp.minimum(
        jnp.sum((m_ids[:, None] >= bounds[None, :]).astype(jnp.int32), axis=1),
        E - 1,
    ).astype(jnp.int32)
    tu = bounds[-1].astype(jnp.int32)
    te = jnp.where(m_ids < tu, te, te[tu - 1])
    sp = jnp.concatenate([te, tu[None]])
    return pp, st, gs, sp


@jax.jit
def kernel(x, Wr, br, W1, b1, W2, b2):
    x_flat = x.reshape(T, D)
    wr_pad = jnp.zeros((D, LANES), jnp.float32).at[:, :E].set(Wr)
    br_pad = jnp.zeros((1, LANES), jnp.float32).at[0, :E].set(br)
    g0, g1, i0, i1 = _router(x_flat, wr_pad, br_pad)
    pp, st, gs, sp = _dispatch_indices(i0, i1, g0, g1)
    sc_gather, sc_combine = _sc_kernels()
    xg = sc_gather(x_flat, st)
    yg = _ffn(sp, xg, W1, b1.reshape(E, 1, DFF), W2, b2.reshape(E, 1, D), gs)
    pp2 = pp.reshape(T, TOPK)
    out_flat = sc_combine(yg, pp2[:, 0], pp2[:, 1])
    return out_flat.reshape(B, S, D)
